# baseline (speedup 1.0000x reference)
import functools

import jax
import jax.numpy as jnp
from jax import lax
from jax.experimental import pallas as pl
from jax.experimental.pallas import tpu as pltpu
from jax.experimental.pallas import tpu_sc as plsc

N = 10000
C = 128
K = 27
NPAD = 10240
NW = 32
RP = NPAD // NW
GCH = 64
NCH = RP // GCH
BN = 1024
NB = NPAD // BN
EPS = 1e-6


def _silu(v):
    return v * jax.nn.sigmoid(v)


def _ss_body(emb_ref, wemb_ref, bemb_ref, ss_ref):
    e = emb_ref[...]
    e = _silu(e)
    ss_ref[...] = (
        jnp.dot(e, wemb_ref[...], preferred_element_type=jnp.float32)
        + bemb_ref[...]
    )


def _e1_body(x_ref, g_ref, b_ref, h_ref):
    i = pl.program_id(0)
    x = x_ref[...]
    mu = jnp.mean(x, axis=-1, keepdims=True)
    var = jnp.mean((x - mu) ** 2, axis=-1, keepdims=True)
    h = (x - mu) * jax.lax.rsqrt(var + EPS)
    h = h * g_ref[...] + b_ref[...]
    h = _silu(h)
    rows = i * BN + lax.broadcasted_iota(jnp.int32, (BN, 1), 0)
    h_ref[...] = jnp.where(rows < N, h, 0.0)


def _e2_body(v_ref, b1_ref, ss_ref, h_ref):
    i = pl.program_id(0)
    v = v_ref[...] + b1_ref[...]
    mu = jnp.mean(v, axis=-1, keepdims=True)
    var = jnp.mean((v - mu) ** 2, axis=-1, keepdims=True)
    h = (v - mu) * jax.lax.rsqrt(var + EPS)
    scale = ss_ref[0, :C]
    shift = ss_ref[0, C:]
    h = h * (1.0 + scale) + shift
    h = _silu(h)
    rows = i * BN + lax.broadcasted_iota(jnp.int32, (BN, 1), 0)
    h_ref[...] = jnp.where(rows < N, h, 0.0)


def _mm_body(h_ref, w_ref, y_ref):
    y_ref[0] = jnp.dot(h_ref[...], w_ref[0],
                       preferred_element_type=jnp.float32)


def _final_body(v_ref, b2_ref, x_ref, o_ref):
    o_ref[...] = v_ref[...] + b2_ref[...] + x_ref[...]


def _tc_ss(emb, w_emb, b_emb):
    return pl.pallas_call(
        _ss_body,
        out_shape=jax.ShapeDtypeStruct((1, 2 * C), jnp.float32),
    )(emb, w_emb, b_emb)


def _tc_e1(xp, gamma, beta):
    return pl.pallas_call(
        _e1_body,
        grid=(NB,),
        in_specs=[
            pl.BlockSpec((BN, C), lambda i: (i, 0)),
            pl.BlockSpec((C,), lambda i: (0,)),
            pl.BlockSpec((C,), lambda i: (0,)),
        ],
        out_specs=pl.BlockSpec((BN, C), lambda i: (i, 0)),
        out_shape=jax.ShapeDtypeStruct((NPAD, C), jnp.float32),
    )(xp, gamma, beta)


def _tc_e2(v, b1, ss):
    return pl.pallas_call(
        _e2_body,
        grid=(NB,),
        in_specs=[
            pl.BlockSpec((BN, C), lambda i: (i, 0)),
            pl.BlockSpec((C,), lambda i: (0,)),
            pl.BlockSpec((1, 2 * C), lambda i: (0, 0)),
        ],
        out_specs=pl.BlockSpec((BN, C), lambda i: (i, 0)),
        out_shape=jax.ShapeDtypeStruct((NPAD, C), jnp.float32),
    )(v, b1, ss)


def _tc_mm(h, w):
    return pl.pallas_call(
        _mm_body,
        grid=(NB, K),
        in_specs=[
            pl.BlockSpec((BN, C), lambda i, k: (i, 0)),
            pl.BlockSpec((1, C, C), lambda i, k: (k, 0, 0)),
        ],
        out_specs=pl.BlockSpec((1, BN, C), lambda i, k: (k, i, 0)),
        out_shape=jax.ShapeDtypeStruct((K, NPAD, C), jnp.float32),
    )(h, w)


def _tc_final(v, b2, x):
    nbf = 10
    bf = N // nbf
    return pl.pallas_call(
        _final_body,
        grid=(nbf,),
        in_specs=[
            pl.BlockSpec((bf, C), lambda i: (i, 0)),
            pl.BlockSpec((C,), lambda i: (0,)),
            pl.BlockSpec((bf, C), lambda i: (i, 0)),
        ],
        out_specs=pl.BlockSpec((bf, C), lambda i: (i, 0)),
        out_shape=jax.ShapeDtypeStruct((N, C), jnp.float32),
    )(v, b2, x)


def _sc_body(y_hbm, nbr_hbm, out_hbm, nbr_v, idx_v, acc_v, sem):
    wid = lax.axis_index("s") * 2 + lax.axis_index("c")
    base = wid * RP

    def compute_idx(k):
        for c in range(NCH):
            for j in range(GCH // 16):
                sl = pl.ds(c * GCH + j * 16, 16)
                idx_v[c, pl.ds(j * 16, 16)] = nbr_v[sl] + k * NPAD

    def fire(add):
        descs = [
            pltpu.async_copy(
                y_hbm.at[idx_v.at[c]],
                acc_v.at[pl.ds(c * GCH, GCH)],
                sem,
                add=add,
            )
            for c in range(NCH)
        ]
        for d in descs:
            d.wait()

    pltpu.sync_copy(nbr_hbm.at[pl.ds(base, RP)], nbr_v)
    compute_idx(0)
    fire(add=False)

    def step(k, _):
        pltpu.sync_copy(nbr_hbm.at[pl.ds(k * NPAD + base, RP)], nbr_v)
        compute_idx(k)
        fire(add=True)
        return ()

    lax.fori_loop(1, K, step, ())
    pltpu.sync_copy(acc_v, out_hbm.at[pl.ds(base, RP)])


_sc_gather_sum = pl.kernel(
    _sc_body,
    out_type=jax.ShapeDtypeStruct((NPAD, C), jnp.float32),
    mesh=plsc.VectorSubcoreMesh(core_axis_name="c", subcore_axis_name="s"),
    scratch_types=[
        pltpu.VMEM((RP,), jnp.int32),
        pltpu.VMEM((NCH, GCH), jnp.int32),
        pltpu.VMEM((RP, C), jnp.float32),
        pltpu.SemaphoreType.DMA,
    ],
)


def kernel(x_feats, emb, nbr_idx, gamma1, beta1, W1, b1, W2, b2, W_emb, b_emb):
    xp = jnp.pad(x_feats, ((0, NPAD - N), (0, 0)))
    nbr_flat = jnp.pad(
        nbr_idx.astype(jnp.int32).T, ((0, 0), (0, NPAD - N)),
        constant_values=N,
    ).reshape(K * NPAD)

    ss = _tc_ss(emb, W_emb, b_emb)

    h1 = _tc_e1(xp, gamma1, beta1)
    y1 = _tc_mm(h1, W1).reshape(K * NPAD, C)
    c1 = _sc_gather_sum(y1, nbr_flat)

    h2 = _tc_e2(c1, b1, ss)
    y2 = _tc_mm(h2, W2).reshape(K * NPAD, C)
    c2 = _sc_gather_sum(y2, nbr_flat)

    return _tc_final(c2[:N], b2, x_feats)


# speedup vs baseline: 1.0637x; 1.0637x over previous
import functools

import jax
import jax.numpy as jnp
from jax import lax
from jax.experimental import pallas as pl
from jax.experimental.pallas import tpu as pltpu
from jax.experimental.pallas import tpu_sc as plsc

N = 10000
C = 128
K = 27
NPAD = 10240
NW = 32
RP = NPAD // NW
GCH = 64
NCH = RP // GCH
BN = 1024
NB = NPAD // BN
EPS = 1e-6


def _silu(v):
    return v * jax.nn.sigmoid(v)


def _ss_body(emb_ref, wemb_ref, bemb_ref, ss_ref):
    e = emb_ref[...]
    e = _silu(e)
    ss_ref[...] = (
        jnp.dot(e, wemb_ref[...], preferred_element_type=jnp.float32)
        + bemb_ref[...]
    )


def _e1_body(x_ref, g_ref, b_ref, h_ref):
    i = pl.program_id(0)
    x = x_ref[...]
    mu = jnp.mean(x, axis=-1, keepdims=True)
    var = jnp.mean((x - mu) ** 2, axis=-1, keepdims=True)
    h = (x - mu) * jax.lax.rsqrt(var + EPS)
    h = h * g_ref[...] + b_ref[...]
    h = _silu(h)
    rows = i * BN + lax.broadcasted_iota(jnp.int32, (BN, 1), 0)
    h_ref[...] = jnp.where(rows < N, h, 0.0)


def _e2_body(v_ref, b1_ref, ss_ref, h_ref):
    i = pl.program_id(0)
    v = v_ref[...] + b1_ref[...]
    mu = jnp.mean(v, axis=-1, keepdims=True)
    var = jnp.mean((v - mu) ** 2, axis=-1, keepdims=True)
    h = (v - mu) * jax.lax.rsqrt(var + EPS)
    scale = ss_ref[0, :C]
    shift = ss_ref[0, C:]
    h = h * (1.0 + scale) + shift
    h = _silu(h)
    rows = i * BN + lax.broadcasted_iota(jnp.int32, (BN, 1), 0)
    h_ref[...] = jnp.where(rows < N, h, 0.0)


def _mm_body(h_ref, w_ref, y_ref):
    y_ref[0] = jnp.dot(h_ref[...], w_ref[0],
                       preferred_element_type=jnp.float32)


def _final_body(v_ref, b2_ref, x_ref, o_ref):
    o_ref[...] = v_ref[...] + b2_ref[...] + x_ref[...]


def _tc_ss(emb, w_emb, b_emb):
    return pl.pallas_call(
        _ss_body,
        out_shape=jax.ShapeDtypeStruct((1, 2 * C), jnp.float32),
    )(emb, w_emb, b_emb)


def _tc_e1(xp, gamma, beta):
    return pl.pallas_call(
        _e1_body,
        grid=(NB,),
        in_specs=[
            pl.BlockSpec((BN, C), lambda i: (i, 0)),
            pl.BlockSpec((C,), lambda i: (0,)),
            pl.BlockSpec((C,), lambda i: (0,)),
        ],
        out_specs=pl.BlockSpec((BN, C), lambda i: (i, 0)),
        out_shape=jax.ShapeDtypeStruct((NPAD, C), jnp.float32),
    )(xp, gamma, beta)


def _tc_e2(v, b1, ss):
    return pl.pallas_call(
        _e2_body,
        grid=(NB,),
        in_specs=[
            pl.BlockSpec((BN, C), lambda i: (i, 0)),
            pl.BlockSpec((C,), lambda i: (0,)),
            pl.BlockSpec((1, 2 * C), lambda i: (0, 0)),
        ],
        out_specs=pl.BlockSpec((BN, C), lambda i: (i, 0)),
        out_shape=jax.ShapeDtypeStruct((NPAD, C), jnp.float32),
    )(v, b1, ss)


def _tc_mm(h, w):
    return pl.pallas_call(
        _mm_body,
        grid=(NB, K),
        in_specs=[
            pl.BlockSpec((BN, C), lambda i, k: (i, 0)),
            pl.BlockSpec((1, C, C), lambda i, k: (k, 0, 0)),
        ],
        out_specs=pl.BlockSpec((1, BN, C), lambda i, k: (k, i, 0)),
        out_shape=jax.ShapeDtypeStruct((K, NPAD, C), jnp.float32),
    )(h, w)


def _tc_final(v, b2, x):
    nbf = 10
    bf = N // nbf
    return pl.pallas_call(
        _final_body,
        grid=(nbf,),
        in_specs=[
            pl.BlockSpec((bf, C), lambda i: (i, 0)),
            pl.BlockSpec((C,), lambda i: (0,)),
            pl.BlockSpec((bf, C), lambda i: (i, 0)),
        ],
        out_specs=pl.BlockSpec((bf, C), lambda i: (i, 0)),
        out_shape=jax.ShapeDtypeStruct((N, C), jnp.float32),
    )(v, b2, x)


def _sc_body(y_hbm, nbr_hbm, out_hbm, nbr_v, idx_v, acc_v, sem, sem_idx):
    wid = lax.axis_index("s") * 2 + lax.axis_index("c")
    base = wid * RP

    nbr_descs = [
        pltpu.async_copy(
            nbr_hbm.at[pl.ds(k * NPAD + base, RP)],
            nbr_v.at[pl.ds(k * RP, RP)],
            sem_idx,
        )
        for k in range(K)
    ]
    for d in nbr_descs:
        d.wait()

    def compute_idx(k, _):
        for j in range(RP // 16):
            sl = pl.ds(k * RP + j * 16, 16)
            idx_v[sl] = nbr_v[sl] + k * NPAD
        return ()

    lax.fori_loop(0, K, compute_idx, ())

    def fire(k, add):
        return [
            pltpu.async_copy(
                y_hbm.at[idx_v.at[pl.ds(k * RP + c * GCH, GCH)]],
                acc_v.at[pl.ds(c * GCH, GCH)],
                sem,
                add=add,
            )
            for c in range(NCH)
        ]

    for d in fire(0, add=False):
        d.wait()

    def launch(k, _):
        fire(k, add=True)
        return ()

    lax.fori_loop(1, K, launch, ())

    def drain(k, _):
        pltpu.make_async_copy(y_hbm.at[pl.ds(0, RP)], acc_v, sem).wait()
        return ()

    lax.fori_loop(1, K, drain, ())

    pltpu.sync_copy(acc_v, out_hbm.at[pl.ds(base, RP)])


_sc_gather_sum = pl.kernel(
    _sc_body,
    out_type=jax.ShapeDtypeStruct((NPAD, C), jnp.float32),
    mesh=plsc.VectorSubcoreMesh(core_axis_name="c", subcore_axis_name="s"),
    scratch_types=[
        pltpu.VMEM((K * RP,), jnp.int32),
        pltpu.VMEM((K * RP,), jnp.int32),
        pltpu.VMEM((RP, C), jnp.float32),
        pltpu.SemaphoreType.DMA,
        pltpu.SemaphoreType.DMA,
    ],
)


def kernel(x_feats, emb, nbr_idx, gamma1, beta1, W1, b1, W2, b2, W_emb, b_emb):
    xp = jnp.pad(x_feats, ((0, NPAD - N), (0, 0)))
    nbr_flat = jnp.pad(
        nbr_idx.astype(jnp.int32).T, ((0, 0), (0, NPAD - N)),
        constant_values=N,
    ).reshape(K * NPAD)

    ss = _tc_ss(emb, W_emb, b_emb)

    h1 = _tc_e1(xp, gamma1, beta1)
    y1 = _tc_mm(h1, W1).reshape(K * NPAD, C)
    c1 = _sc_gather_sum(y1, nbr_flat)

    h2 = _tc_e2(c1, b1, ss)
    y2 = _tc_mm(h2, W2).reshape(K * NPAD, C)
    c2 = _sc_gather_sum(y2, nbr_flat)

    return _tc_final(c2[:N], b2, x_feats)


# speedup vs baseline: 3.7641x; 3.5388x over previous
import functools

import jax
import jax.numpy as jnp
from jax import lax
from jax.experimental import pallas as pl
from jax.experimental.pallas import tpu as pltpu
from jax.experimental.pallas import tpu_sc as plsc

N = 10000
C = 128
K = 27
NPAD = 10240
NW = 32
RP = NPAD // NW
GCH = 64
NCH = RP // GCH
BN = 1024
NB = NPAD // BN
EPS = 1e-6


def _silu(v):
    return v * jax.nn.sigmoid(v)


def _ss_body(emb_ref, wemb_ref, bemb_ref, ss_ref):
    e = emb_ref[...]
    e = _silu(e)
    ss_ref[...] = (
        jnp.dot(e, wemb_ref[...], preferred_element_type=jnp.float32)
        + bemb_ref[...]
    )


def _e1_body(x_ref, g_ref, b_ref, h_ref):
    i = pl.program_id(0)
    x = x_ref[...]
    mu = jnp.mean(x, axis=-1, keepdims=True)
    var = jnp.mean((x - mu) ** 2, axis=-1, keepdims=True)
    h = (x - mu) * jax.lax.rsqrt(var + EPS)
    h = h * g_ref[...] + b_ref[...]
    h = _silu(h)
    rows = i * BN + lax.broadcasted_iota(jnp.int32, (BN, 1), 0)
    h_ref[...] = jnp.where(rows < N, h, 0.0)


def _e2_body(v_ref, b1_ref, ss_ref, h_ref):
    i = pl.program_id(0)
    v = v_ref[...] + b1_ref[...]
    mu = jnp.mean(v, axis=-1, keepdims=True)
    var = jnp.mean((v - mu) ** 2, axis=-1, keepdims=True)
    h = (v - mu) * jax.lax.rsqrt(var + EPS)
    scale = ss_ref[0, :C]
    shift = ss_ref[0, C:]
    h = h * (1.0 + scale) + shift
    h = _silu(h)
    rows = i * BN + lax.broadcasted_iota(jnp.int32, (BN, 1), 0)
    h_ref[...] = jnp.where(rows < N, h, 0.0)


def _mm_body(h_ref, w_ref, y_ref):
    y_ref[0] = jnp.dot(h_ref[...], w_ref[0],
                       preferred_element_type=jnp.float32)


def _final_body(v_ref, b2_ref, x_ref, o_ref):
    o_ref[...] = v_ref[...] + b2_ref[...] + x_ref[...]


def _tc_ss(emb, w_emb, b_emb):
    return pl.pallas_call(
        _ss_body,
        out_shape=jax.ShapeDtypeStruct((1, 2 * C), jnp.float32),
    )(emb, w_emb, b_emb)


def _tc_e1(xp, gamma, beta):
    return pl.pallas_call(
        _e1_body,
        grid=(NB,),
        in_specs=[
            pl.BlockSpec((BN, C), lambda i: (i, 0)),
            pl.BlockSpec((C,), lambda i: (0,)),
            pl.BlockSpec((C,), lambda i: (0,)),
        ],
        out_specs=pl.BlockSpec((BN, C), lambda i: (i, 0)),
        out_shape=jax.ShapeDtypeStruct((NPAD, C), jnp.float32),
    )(xp, gamma, beta)


def _tc_e2(v, b1, ss):
    return pl.pallas_call(
        _e2_body,
        grid=(NB,),
        in_specs=[
            pl.BlockSpec((BN, C), lambda i: (i, 0)),
            pl.BlockSpec((C,), lambda i: (0,)),
            pl.BlockSpec((1, 2 * C), lambda i: (0, 0)),
        ],
        out_specs=pl.BlockSpec((BN, C), lambda i: (i, 0)),
        out_shape=jax.ShapeDtypeStruct((NPAD, C), jnp.float32),
    )(v, b1, ss)


def _tc_mm(h, w):
    return pl.pallas_call(
        _mm_body,
        grid=(NB, K),
        in_specs=[
            pl.BlockSpec((BN, C), lambda i, k: (i, 0)),
            pl.BlockSpec((1, C, C), lambda i, k: (k, 0, 0)),
        ],
        out_specs=pl.BlockSpec((1, BN, C), lambda i, k: (k, i, 0)),
        out_shape=jax.ShapeDtypeStruct((K, NPAD, C), jnp.float32),
    )(h, w)


def _tc_final(v, b2, x):
    nbf = 10
    bf = N // nbf
    return pl.pallas_call(
        _final_body,
        grid=(nbf,),
        in_specs=[
            pl.BlockSpec((bf, C), lambda i: (i, 0)),
            pl.BlockSpec((C,), lambda i: (0,)),
            pl.BlockSpec((bf, C), lambda i: (i, 0)),
        ],
        out_specs=pl.BlockSpec((bf, C), lambda i: (i, 0)),
        out_shape=jax.ShapeDtypeStruct((N, C), jnp.float32),
    )(v, b2, x)


def _sc_body(y_hbm, nbr_hbm, out_hbm, spm, nbr_v, acc_v, sem, sem2):
    cid = lax.axis_index("c")
    sid = lax.axis_index("s")
    base = (sid * 2 + cid) * RP
    srows = 632

    nbr_descs = [
        pltpu.async_copy(
            nbr_hbm.at[pl.ds(k * NPAD + base, RP)],
            nbr_v.at[pl.ds(k * RP, RP)],
            sem2,
        )
        for k in range(K)
    ]
    for d in nbr_descs:
        d.wait()

    def stage(k):
        pltpu.async_copy(
            y_hbm.at[pl.ds(k * NPAD + sid * srows, srows)],
            spm.at[pl.ds(sid * srows, srows)],
            sem2,
        ).wait()
        plsc.subcore_barrier()

    def gather(k, add):
        descs = [
            pltpu.async_copy(
                spm.at[nbr_v.at[pl.ds(k * RP + c * GCH, GCH)]],
                acc_v.at[pl.ds(c * GCH, GCH)],
                sem,
                add=add,
            )
            for c in range(NCH)
        ]
        for d in descs:
            d.wait()
        plsc.subcore_barrier()

    stage(0)
    gather(0, add=False)

    def step(k, _):
        stage(k)
        gather(k, add=True)
        return ()

    lax.fori_loop(1, K, step, ())

    pltpu.sync_copy(acc_v, out_hbm.at[pl.ds(base, RP)])


_sc_gather_sum = pl.kernel(
    _sc_body,
    out_type=jax.ShapeDtypeStruct((NPAD, C), jnp.float32),
    mesh=plsc.VectorSubcoreMesh(core_axis_name="c", subcore_axis_name="s"),
    scratch_types=[
        pltpu.VMEM_SHARED((16 * 632, C), jnp.float32),
        pltpu.VMEM((K * RP,), jnp.int32),
        pltpu.VMEM((RP, C), jnp.float32),
        pltpu.SemaphoreType.DMA,
        pltpu.SemaphoreType.DMA,
    ],
)


def kernel(x_feats, emb, nbr_idx, gamma1, beta1, W1, b1, W2, b2, W_emb, b_emb):
    xp = jnp.pad(x_feats, ((0, NPAD - N), (0, 0)))
    nbr_flat = jnp.pad(
        nbr_idx.astype(jnp.int32).T, ((0, 0), (0, NPAD - N)),
        constant_values=N,
    ).reshape(K * NPAD)

    ss = _tc_ss(emb, W_emb, b_emb)

    h1 = _tc_e1(xp, gamma1, beta1)
    y1 = _tc_mm(h1, W1).reshape(K * NPAD, C)
    c1 = _sc_gather_sum(y1, nbr_flat)

    h2 = _tc_e2(c1, b1, ss)
    y2 = _tc_mm(h2, W2).reshape(K * NPAD, C)
    c2 = _sc_gather_sum(y2, nbr_flat)

    return _tc_final(c2[:N], b2, x_feats)
